# baseline (speedup 1.0000x reference)
import functools

import jax
import jax.numpy as jnp
from jax import lax
from jax.experimental import pallas as pl
from jax.experimental.pallas import tpu as pltpu
from jax.experimental.pallas import tpu_sc as plsc

_TOPK = 8
_N_GROUPS = 8
_TOPK_GROUPS = 4
_ROUTE_SCALE = 2.5
_N_EXPERTS = 64
_DIM = 768
_TOKENS = 32768

_N_CHUNKS = 4
_CT = _TOKENS // _N_CHUNKS
_NW = 32
_CHUNK = _CT // _NW
_BTC = 1024
_L = 16


def _mm_block(x_ref, w_ref, o_ref):
    st = lax.dot_general(
        w_ref[...], x_ref[...], (((1,), (1,)), ((), ())),
        preferred_element_type=jnp.float32,
    )
    st = jax.nn.sigmoid(st)
    for b in range(_BTC // _CHUNK):
        o_ref[b] = st[:, b * _CHUNK:(b + 1) * _CHUNK]


def _scores_chunked(x, weight):
    return pl.pallas_call(
        _mm_block,
        grid=(_CT // _BTC,),
        in_specs=[
            pl.BlockSpec((_BTC, _DIM), lambda i: (i, 0)),
            pl.BlockSpec((_N_EXPERTS, _DIM), lambda i: (0, 0)),
        ],
        out_specs=pl.BlockSpec(
            (_BTC // _CHUNK, _N_EXPERTS, _CHUNK), lambda i: (i, 0, 0)
        ),
        out_shape=jax.ShapeDtypeStruct((_NW, _N_EXPERTS, _CHUNK), jnp.float32),
        compiler_params=pltpu.CompilerParams(dimension_semantics=("arbitrary",)),
    )(x, weight)


@functools.partial(
    pl.kernel,
    mesh=plsc.VectorSubcoreMesh(core_axis_name="c", subcore_axis_name="s"),
    out_type=[
        jax.ShapeDtypeStruct((_NW, _TOPK, _CHUNK), jnp.float32),
        jax.ShapeDtypeStruct((_NW, _TOPK, _CHUNK), jnp.int32),
    ],
    scratch_types=[
        pltpu.VMEM((_N_EXPERTS, _CHUNK), jnp.float32),
        pltpu.VMEM((_TOPK, _CHUNK), jnp.float32),
        pltpu.VMEM((_TOPK, _CHUNK), jnp.int32),
    ],
)
def _route_sc(scores_hbm, wout_hbm, iout_hbm, s_v, w_v, i_v):
    wid = lax.axis_index("s") * 2 + lax.axis_index("c")
    pltpu.sync_copy(scores_hbm.at[wid], s_v)

    def body(j, carry):
        off = j * _L
        s = [s_v[e, pl.ds(off, _L)] for e in range(_N_EXPERTS)]

        gs = []
        for g in range(_N_GROUPS):
            b = 8 * g
            m1, m2 = [], []
            for p in range(4):
                a, c = s[b + 2 * p], s[b + 2 * p + 1]
                m1.append(jnp.maximum(a, c))
                m2.append(jnp.minimum(a, c))
            while len(m1) > 1:
                n1, n2 = [], []
                for p in range(0, len(m1), 2):
                    n1.append(jnp.maximum(m1[p], m1[p + 1]))
                    n2.append(
                        jnp.maximum(
                            jnp.minimum(m1[p], m1[p + 1]),
                            jnp.maximum(m2[p], m2[p + 1]),
                        )
                    )
                m1, m2 = n1, n2
            gs.append(m1[0] + m2[0])

        one = jnp.full((_L,), 1, jnp.int32)
        zero = jnp.full((_L,), 0, jnp.int32)
        ge = {}
        for a in range(_N_GROUPS):
            for c in range(a + 1, _N_GROUPS):
                ge[(a, c)] = jnp.where(gs[a] >= gs[c], one, zero)
        sel = []
        for g in range(_N_GROUPS):
            r = zero
            for a in range(g):
                r = r + ge[(a, g)]
            for c in range(g + 1, _N_GROUPS):
                r = r + 1 - ge[(g, c)]
            sel.append(r < _TOPK_GROUPS)

        negv = jnp.full((_L,), float("-inf"), jnp.float32)
        m = [jnp.where(sel[e // 8], s[e], negv) for e in range(_N_EXPERTS)]

        vals, idxs = [], []
        for k in range(_TOPK):
            tv = list(m)
            ti = [jnp.full((_L,), e, jnp.int32) for e in range(_N_EXPERTS)]
            while len(tv) > 1:
                nv, ni = [], []
                for p in range(0, len(tv), 2):
                    cond = tv[p] >= tv[p + 1]
                    nv.append(jnp.where(cond, tv[p], tv[p + 1]))
                    ni.append(jnp.where(cond, ti[p], ti[p + 1]))
                tv, ti = nv, ni
            vals.append(tv[0])
            idxs.append(ti[0])
            if k + 1 < _TOPK:
                m = [
                    jnp.where(ti[0] == e, negv, m[e])
                    for e in range(_N_EXPERTS)
                ]

        tot = vals[0]
        for k in range(1, _TOPK):
            tot = tot + vals[k]
        scale = _ROUTE_SCALE / tot
        for k in range(_TOPK):
            w_v[k, pl.ds(off, _L)] = vals[k] * scale
            i_v[k, pl.ds(off, _L)] = idxs[k]
        return carry

    lax.fori_loop(0, _CHUNK // _L, body, 0)
    pltpu.sync_copy(w_v, wout_hbm.at[wid])
    pltpu.sync_copy(i_v, iout_hbm.at[wid])


@jax.jit
def kernel(x, weight):
    wparts, iparts = [], []
    for c in range(_N_CHUNKS):
        scores = _scores_chunked(x[c * _CT:(c + 1) * _CT], weight)
        wc, ic = _route_sc(scores)
        wparts.append(wc)
        iparts.append(ic)
    wts = jnp.concatenate(
        [w.transpose(0, 2, 1).reshape(_CT, _TOPK) for w in wparts], axis=0
    )
    idx = jnp.concatenate(
        [i.transpose(0, 2, 1).reshape(_CT, _TOPK) for i in iparts], axis=0
    )
    return wts, idx


# speedup vs baseline: 2.7704x; 2.7704x over previous
import jax
import jax.numpy as jnp
from jax.experimental import pallas as pl
from jax.experimental.pallas import tpu as pltpu

_TOPK = 8
_N_GROUPS = 8
_TOPK_GROUPS = 4
_ROUTE_SCALE = 2.5
_N_EXPERTS = 64
_DIM = 768
_TOKENS = 32768

_BT = 4096
_NEG = float("-inf")


def _moe_gate_block(x_ref, w_ref, wout_ref, iout_ref):
    x = x_ref[...]
    w = w_ref[...]
    st = jax.lax.dot_general(
        w, x, (((1,), (1,)), ((), ())), preferred_element_type=jnp.float32
    )
    st = jax.nn.sigmoid(st)

    row = jax.lax.broadcasted_iota(jnp.int32, (_N_EXPERTS, _BT), 0)

    m1 = st
    m2 = None
    for k in (1, 2, 4):
        bit = (row & k) == 0
        pm1 = jnp.where(bit, pltpu.roll(m1, _N_EXPERTS - k, 0), pltpu.roll(m1, k, 0))
        if m2 is None:
            m2 = jnp.minimum(m1, pm1)
        else:
            pm2 = jnp.where(bit, pltpu.roll(m2, _N_EXPERTS - k, 0), pltpu.roll(m2, k, 0))
            m2 = jnp.maximum(jnp.minimum(m1, pm1), jnp.maximum(m2, pm2))
        m1 = jnp.maximum(m1, pm1)
    gs = m1 + m2

    g = row >> 3
    rank = jnp.zeros((_N_EXPERTS, _BT), dtype=jnp.int32)
    for j in range(1, _N_GROUPS):
        other = pltpu.roll(gs, _N_EXPERTS - 8 * j, 0)
        og_lt = ((g + j) & 7) < g
        beats = (other > gs) | ((other == gs) & og_lt)
        rank = rank + jnp.where(beats, 1, 0)
    sel = rank < _TOPK_GROUPS

    masked = jnp.where(sel, st, _NEG)
    picked_v, picked_i = [], []
    for _ in range(_TOPK):
        m = jnp.max(masked, axis=0, keepdims=True)
        am = jnp.min(
            jnp.where(masked == m, row, _N_EXPERTS), axis=0, keepdims=True
        )
        picked_v.append(m)
        picked_i.append(am)
        if len(picked_v) < _TOPK:
            masked = jnp.where(row == am, _NEG, masked)

    vals = jnp.concatenate(picked_v, axis=0)
    idx = jnp.concatenate(picked_i, axis=0)
    wts = vals * (_ROUTE_SCALE / jnp.sum(vals, axis=0, keepdims=True))

    wout_ref[...] = wts
    iout_ref[...] = idx


@jax.jit
def kernel(x, weight):
    grid = (_TOKENS // _BT,)
    wout, iout = pl.pallas_call(
        _moe_gate_block,
        grid=grid,
        in_specs=[
            pl.BlockSpec((_BT, _DIM), lambda i: (i, 0)),
            pl.BlockSpec((_N_EXPERTS, _DIM), lambda i: (0, 0)),
        ],
        out_specs=[
            pl.BlockSpec((_TOPK, _BT), lambda i: (0, i)),
            pl.BlockSpec((_TOPK, _BT), lambda i: (0, i)),
        ],
        out_shape=[
            jax.ShapeDtypeStruct((_TOPK, _TOKENS), jnp.float32),
            jax.ShapeDtypeStruct((_TOPK, _TOKENS), jnp.int32),
        ],
        compiler_params=pltpu.CompilerParams(
            dimension_semantics=("arbitrary",),
        ),
    )(x, weight)
    return wout.T, iout.T
